# baseline (speedup 1.0000x reference)
import functools
import math

import jax
import jax.numpy as jnp
from jax import lax
from jax.experimental import pallas as pl
from jax.experimental.pallas import tpu as pltpu
from jax.experimental.pallas import tpu_sc as plsc


def _make_embed_kernel(n_total, d_model, n_per_w, chunk, nbuf, num_cores, scale):
    n_chunks = n_per_w // chunk
    n_outer = n_chunks // nbuf
    mesh = plsc.VectorSubcoreMesh(core_axis_name="c", subcore_axis_name="s")

    scratch = [pltpu.VMEM((n_per_w,), jnp.int32)]
    scratch += [pltpu.VMEM((chunk, d_model), jnp.float32) for _ in range(2 * nbuf)]
    scratch += [pltpu.SemaphoreType.DMA for _ in range(2 * nbuf)]

    @functools.partial(
        pl.kernel,
        mesh=mesh,
        out_type=jax.ShapeDtypeStruct((n_total, d_model), jnp.float32),
        compiler_params=pltpu.CompilerParams(use_tc_tiling_on_sc=False),
        scratch_types=scratch,
    )
    def k(idx_hbm, table_hbm, out_hbm, idx_v, *rest):
        rows = rest[:nbuf]
        obuf = rest[nbuf : 2 * nbuf]
        gsem = rest[2 * nbuf : 3 * nbuf]
        osem = rest[3 * nbuf : 4 * nbuf]
        wid = lax.axis_index("s") * num_cores + lax.axis_index("c")
        base = wid * n_per_w
        pltpu.sync_copy(idx_hbm.at[pl.ds(base, n_per_w)], idx_v)

        def g_start(c, b):
            pltpu.make_async_copy(
                table_hbm.at[idx_v.at[pl.ds(c * chunk, chunk)]], rows[b], gsem[b]
            ).start()

        def g_wait(c, b):
            pltpu.make_async_copy(
                table_hbm.at[idx_v.at[pl.ds(c * chunk, chunk)]], rows[b], gsem[b]
            ).wait()

        def o_start(c, b):
            pltpu.make_async_copy(
                obuf[b], out_hbm.at[pl.ds(base + c * chunk, chunk)], osem[b]
            ).start()

        def o_wait(c, b):
            pltpu.make_async_copy(
                obuf[b], out_hbm.at[pl.ds(base + c * chunk, chunk)], osem[b]
            ).wait()

        for b in range(nbuf):
            g_start(b, b)

        def outer(g2, _):
            for b in range(nbuf):
                c = g2 * nbuf + b
                g_wait(c, b)

                @pl.when(g2 > 0)
                def _():
                    o_wait(c, b)

                def row_body(j, _):
                    for u in range(2):
                        jj = j * 2 + u
                        for t in range(d_model // 16):
                            sl = pl.ds(t * 16, 16)
                            obuf[b][jj, sl] = rows[b][jj, sl] * scale
                    return 0

                lax.fori_loop(0, chunk // 2, row_body, 0, unroll=2)
                o_start(c, b)

                @pl.when(c + nbuf < n_chunks)
                def _():
                    g_start(c + nbuf, b)

            return 0

        lax.fori_loop(0, n_outer, outer, 0)
        for b in range(nbuf):
            o_wait(n_chunks - nbuf + b, b)

    return k


def kernel(x, table):
    b, l = x.shape
    v, d = table.shape
    n_total = b * l
    idx = x.reshape(n_total).astype(jnp.int32)
    info = plsc.get_sparse_core_info()
    nw = info.num_cores * info.num_subcores
    n_per_w = n_total // nw
    k = _make_embed_kernel(
        n_total, d, n_per_w, 128, 2, info.num_cores, float(math.sqrt(d))
    )
    out = k(idx, table)
    return out.reshape(b, l, d)
